# baseline (speedup 1.0000x reference)
import functools

import jax
import jax.numpy as jnp
from jax import lax
from jax.experimental import pallas as pl
from jax.experimental.pallas import tpu as pltpu
from jax.experimental.pallas import tpu_sc as plsc

T, D, I, E, K = 4096, 2048, 1024, 16, 2
A = T * K
B = 256
P = A + E * B
NB = P // B
J = 8
TI = I // J

NC, NS = 2, 16
NW = NC * NS
TPW = T // NW
RT = 16

RW = 16
CPW = A // RW
CV = CPW // 16


def _splat(vec, idx):
    return vec.at[jnp.full((16,), idx, jnp.int32)].get(mode="promise_in_bounds")


def _routing_body(e_hbm, pos_hbm, be_hbm, nab_hbm,
                  ev_v, rank_v, slot_v, cnt_v, allcnt_v, bev_v, shared_v):
    wid = lax.axis_index("s")
    base_a = wid * CPW
    pltpu.sync_copy(e_hbm.at[pl.ds(base_a, CPW)], ev_v)

    for e in range(E):
        def vec_body(i, base):
            sl = pl.ds(i * 16, 16)
            ev = ev_v[sl]
            m = ev == e
            csum = plsc.cumsum(jnp.where(m, 1, 0))
            rank_v[sl] = jnp.where(m, base + csum - 1, rank_v[sl])
            return base + plsc.all_reduce_population_count(m)

        cnt = lax.fori_loop(0, CV, vec_body, jnp.zeros((16,), jnp.int32))
        lane = lax.iota(jnp.int32, 16)
        cnt_v[...] = jnp.where(lane == e, cnt, cnt_v[...])

    pltpu.sync_copy(cnt_v, shared_v.at[wid])
    plsc.subcore_barrier()
    pltpu.sync_copy(shared_v, allcnt_v)

    gc = jnp.zeros((16,), jnp.int32)
    for w in range(RW):
        gc = gc + allcnt_v[w]
    padded = (gc + (B - 1)) & ~(B - 1)
    ends = plsc.cumsum(padded)
    base_w = ends - padded
    for w in range(RW):
        base_w = base_w + jnp.where(w < wid, allcnt_v[w], 0)

    def slot_body(i, _):
        sl = pl.ds(i * 16, 16)
        ev = ev_v[sl]
        bases = base_w.at[ev].get(mode="promise_in_bounds")
        slot_v[sl] = bases + rank_v[sl]
        return 0

    lax.fori_loop(0, CV, slot_body, 0)
    pltpu.sync_copy(slot_v, pos_hbm.at[pl.ds(base_a, CPW)])

    @pl.when(wid == 0)
    def _():
        for q in range(NB // 16):
            bs = (lax.iota(jnp.int32, 16) + q * 16) * B
            bev = jnp.zeros((16,), jnp.int32)
            for e in range(E):
                bev = bev + jnp.where(bs >= _splat(ends, e), 1, 0)
            bev_v[pl.ds(q * 16, 16)] = jnp.minimum(bev, E - 1)
        pltpu.sync_copy(bev_v, be_hbm)
        cnt_v[...] = lax.shift_right_logical(_splat(ends, E - 1), B.bit_length() - 1)
        pltpu.sync_copy(cnt_v, nab_hbm)


_route = functools.partial(
    pl.kernel,
    _routing_body,
    out_type=(jax.ShapeDtypeStruct((A,), jnp.int32),
              jax.ShapeDtypeStruct((NB,), jnp.int32),
              jax.ShapeDtypeStruct((16,), jnp.int32)),
    mesh=plsc.VectorSubcoreMesh(core_axis_name="c", subcore_axis_name="s",
                                num_cores=1),
    compiler_params=pltpu.CompilerParams(needs_layout_passes=False),
    scratch_types=[
        pltpu.VMEM((CPW,), jnp.int32),
        pltpu.VMEM((CPW,), jnp.int32),
        pltpu.VMEM((CPW,), jnp.int32),
        pltpu.VMEM((16,), jnp.int32),
        pltpu.VMEM((RW, 16), jnp.int32),
        pltpu.VMEM((NB,), jnp.int32),
        pltpu.HBM((RW, 16), jnp.int32),
    ],
)()


NCH_D = TPW // RT


def _dispatch_body(hid_hbm, pos_hbm, w_hbm, xg_hbm, sw_hbm,
                   rows_v, idx_v, w_v, semf, sems):
    wid = lax.axis_index("s") * NC + lax.axis_index("c")
    tw = wid * TPW

    def fetch(c, b):
        tb = tw + c * RT
        return [
            pltpu.async_copy(pos_hbm.at[pl.ds(tb, RT)], idx_v.at[b, 0], semf),
            pltpu.async_copy(pos_hbm.at[pl.ds(T + tb, RT)], idx_v.at[b, 1], semf),
            pltpu.async_copy(w_hbm.at[pl.ds(tb, RT)], w_v.at[b, 0], semf),
            pltpu.async_copy(w_hbm.at[pl.ds(T + tb, RT)], w_v.at[b, 1], semf),
            pltpu.async_copy(hid_hbm.at[pl.ds(tb, RT)], rows_v.at[b], semf),
        ]

    def scatter(c, b):
        cps = []
        for k in range(K):
            cps.append(pltpu.async_copy(
                rows_v.at[b], xg_hbm.at[idx_v.at[b, k]], sems))
            cps.append(pltpu.async_copy(
                w_v.at[b, k], sw_hbm.at[idx_v.at[b, k]], sems))
        return cps

    fh = {0: fetch(0, 0)}
    sh = {}
    for c in range(NCH_D):
        if c >= 2:
            for cp in sh[c - 2]:
                cp.wait()
        if c + 1 < NCH_D:
            fh[c + 1] = fetch(c + 1, (c + 1) % 3)
        for cp in fh[c]:
            cp.wait()
        sh[c] = scatter(c, c % 3)
    for cp in sh[NCH_D - 2] + sh[NCH_D - 1]:
        cp.wait()


_dispatch = functools.partial(
    pl.kernel,
    _dispatch_body,
    out_type=(jax.ShapeDtypeStruct((P, D), jnp.float32),
              jax.ShapeDtypeStruct((P,), jnp.float32)),
    mesh=plsc.VectorSubcoreMesh(core_axis_name="c", subcore_axis_name="s"),
    scratch_types=[
        pltpu.VMEM((3, RT, D), jnp.float32),
        pltpu.VMEM((3, K, RT), jnp.int32),
        pltpu.VMEM((3, K, RT), jnp.float32),
        pltpu.SemaphoreType.DMA,
        pltpu.SemaphoreType.DMA,
    ],
)()


def _glu_body(be_ref, nab_ref, x_ref, gu_ref, d_ref, w_ref, y_ref):
    @pl.when(pl.program_id(0) < nab_ref[0])
    def _():
        x = x_ref[...].astype(jnp.bfloat16)
        w = w_ref[0, 0][:, None]
        acc = jnp.zeros((B, D), jnp.float32)
        for j in range(J):
            g = lax.dot_general(x, gu_ref[0, pl.ds(j * TI, TI), :].astype(jnp.bfloat16),
                                (((1,), (1,)), ((), ())),
                                preferred_element_type=jnp.float32)
            u = lax.dot_general(x, gu_ref[0, pl.ds(I + j * TI, TI), :].astype(jnp.bfloat16),
                                (((1,), (1,)), ((), ())),
                                preferred_element_type=jnp.float32)
            a = (g * lax.logistic(g) * u * w).astype(jnp.bfloat16)
            acc += lax.dot_general(a, d_ref[0, :, pl.ds(j * TI, TI)].astype(jnp.bfloat16),
                                   (((1,), (1,)), ((), ())),
                                   preferred_element_type=jnp.float32)
        y_ref[...] = acc


def _grouped_glu(xg, gate_up_proj, down_proj, block_expert, nab, slot_weight):
    def bc(b, nab_r):
        return jnp.minimum(b, nab_r[0] - 1)

    grid_spec = pltpu.PrefetchScalarGridSpec(
        num_scalar_prefetch=2,
        grid=(NB,),
        in_specs=[
            pl.BlockSpec((B, D), lambda b, be, na: (bc(b, na), 0)),
            pl.BlockSpec((1, 2 * I, D), lambda b, be, na: (be[bc(b, na)], 0, 0)),
            pl.BlockSpec((1, D, I), lambda b, be, na: (be[bc(b, na)], 0, 0)),
            pl.BlockSpec((1, 1, B), lambda b, be, na: (bc(b, na), 0, 0)),
        ],
        out_specs=pl.BlockSpec((B, D), lambda b, be, na: (bc(b, na), 0)),
    )
    return pl.pallas_call(
        _glu_body,
        grid_spec=grid_spec,
        out_shape=jax.ShapeDtypeStruct((P, D), jnp.float32),
    )(block_expert, nab, xg, gate_up_proj, down_proj, slot_weight)


RTC = 8
NCH_C = TPW // RTC


def _combine_body(y_hbm, pos_hbm, out_hbm,
                  y0_v, y1_v, o_v, idx_v, semg, semo):
    wid = lax.axis_index("s") * NC + lax.axis_index("c")
    tw = wid * TPW
    pltpu.sync_copy(pos_hbm.at[pl.ds(tw, TPW)], idx_v.at[0])
    pltpu.sync_copy(pos_hbm.at[pl.ds(T + tw, TPW)], idx_v.at[1])

    def gath(c, b):
        sl = pl.ds(c * RTC, RTC)
        return [
            pltpu.async_copy(y_hbm.at[idx_v.at[0, sl]], y0_v.at[b], semg),
            pltpu.async_copy(y_hbm.at[idx_v.at[1, sl]], y1_v.at[b], semg),
        ]

    gh = {0: gath(0, 0)}
    oh = {}
    for c in range(NCH_C):
        if c + 1 < NCH_C:
            gh[c + 1] = gath(c + 1, (c + 1) % 2)
        for cp in gh[c]:
            cp.wait()
        if c >= 2:
            oh[c - 2].wait()
        b = c % 2

        def token_body(t, _):
            def col_body(cc, _):
                sl = pl.ds(cc * 16, 16)
                o_v[b, t, sl] = y0_v[b, t, sl] + y1_v[b, t, sl]
                return 0

            lax.fori_loop(0, D // 16, col_body, 0, unroll=8)
            return 0

        lax.fori_loop(0, RTC, token_body, 0)
        oh[c] = pltpu.async_copy(o_v.at[b], out_hbm.at[pl.ds(tw + c * RTC, RTC)],
                                 semo)
    oh[NCH_C - 2].wait()
    oh[NCH_C - 1].wait()


_combine = functools.partial(
    pl.kernel,
    _combine_body,
    out_type=jax.ShapeDtypeStruct((T, D), jnp.float32),
    mesh=plsc.VectorSubcoreMesh(core_axis_name="c", subcore_axis_name="s"),
    scratch_types=[
        pltpu.VMEM((2, RTC, D), jnp.float32),
        pltpu.VMEM((2, RTC, D), jnp.float32),
        pltpu.VMEM((2, RTC, D), jnp.float32),
        pltpu.VMEM((K, TPW), jnp.int32),
        pltpu.SemaphoreType.DMA,
        pltpu.SemaphoreType.DMA,
    ],
)()


@jax.jit
def kernel(hidden_states, top_k_indices, top_k_weights, gate_up_proj, down_proj):
    eT = top_k_indices.T.reshape(-1).astype(jnp.int32)
    wT = top_k_weights.T.reshape(-1)
    pos, block_expert, nab = _route(eT)
    xg, sw = _dispatch(hidden_states, pos, wT)
    y = _grouped_glu(xg, gate_up_proj, down_proj, block_expert, nab,
                     sw.reshape(NB, 1, B))
    return _combine(y, pos)


# speedup vs baseline: 1.2381x; 1.2381x over previous
import functools

import jax
import jax.numpy as jnp
from jax import lax
from jax.experimental import pallas as pl
from jax.experimental.pallas import tpu as pltpu
from jax.experimental.pallas import tpu_sc as plsc

T, D, I, E, K = 4096, 2048, 1024, 16, 2
A = T * K
B = 256
P = A + E * B
NB = P // B
J = 1
TI = I // J

NC, NS = 2, 16
NW = NC * NS
TPW = T // NW
RT = 16

RW = 16
CPW = A // RW
CV = CPW // 16


def _splat(vec, idx):
    return vec.at[jnp.full((16,), idx, jnp.int32)].get(mode="promise_in_bounds")


def _routing_body(e_hbm, pos_hbm, be_hbm, nab_hbm,
                  ev_v, rank_v, slot_v, cnt_v, allcnt_v, bev_v, shared_v):
    wid = lax.axis_index("s")
    base_a = wid * CPW
    pltpu.sync_copy(e_hbm.at[pl.ds(base_a, CPW)], ev_v)

    for e in range(E):
        def vec_body(i, base):
            sl = pl.ds(i * 16, 16)
            ev = ev_v[sl]
            m = ev == e
            csum = plsc.cumsum(jnp.where(m, 1, 0))
            rank_v[sl] = jnp.where(m, base + csum - 1, rank_v[sl])
            return base + plsc.all_reduce_population_count(m)

        cnt = lax.fori_loop(0, CV, vec_body, jnp.zeros((16,), jnp.int32))
        lane = lax.iota(jnp.int32, 16)
        cnt_v[...] = jnp.where(lane == e, cnt, cnt_v[...])

    pltpu.sync_copy(cnt_v, shared_v.at[wid])
    plsc.subcore_barrier()
    pltpu.sync_copy(shared_v, allcnt_v)

    gc = jnp.zeros((16,), jnp.int32)
    for w in range(RW):
        gc = gc + allcnt_v[w]
    padded = (gc + (B - 1)) & ~(B - 1)
    ends = plsc.cumsum(padded)
    base_w = ends - padded
    for w in range(RW):
        base_w = base_w + jnp.where(w < wid, allcnt_v[w], 0)

    def slot_body(i, _):
        sl = pl.ds(i * 16, 16)
        ev = ev_v[sl]
        bases = base_w.at[ev].get(mode="promise_in_bounds")
        slot_v[sl] = bases + rank_v[sl]
        return 0

    lax.fori_loop(0, CV, slot_body, 0)
    pltpu.sync_copy(slot_v, pos_hbm.at[pl.ds(base_a, CPW)])

    @pl.when(wid == 0)
    def _():
        for q in range(NB // 16):
            bs = (lax.iota(jnp.int32, 16) + q * 16) * B
            bev = jnp.zeros((16,), jnp.int32)
            for e in range(E):
                bev = bev + jnp.where(bs >= _splat(ends, e), 1, 0)
            bev_v[pl.ds(q * 16, 16)] = jnp.minimum(bev, E - 1)
        pltpu.sync_copy(bev_v, be_hbm)
        cnt_v[...] = lax.shift_right_logical(_splat(ends, E - 1), B.bit_length() - 1)
        pltpu.sync_copy(cnt_v, nab_hbm)


_route = functools.partial(
    pl.kernel,
    _routing_body,
    out_type=(jax.ShapeDtypeStruct((A,), jnp.int32),
              jax.ShapeDtypeStruct((NB,), jnp.int32),
              jax.ShapeDtypeStruct((16,), jnp.int32)),
    mesh=plsc.VectorSubcoreMesh(core_axis_name="c", subcore_axis_name="s",
                                num_cores=1),
    compiler_params=pltpu.CompilerParams(needs_layout_passes=False),
    scratch_types=[
        pltpu.VMEM((CPW,), jnp.int32),
        pltpu.VMEM((CPW,), jnp.int32),
        pltpu.VMEM((CPW,), jnp.int32),
        pltpu.VMEM((16,), jnp.int32),
        pltpu.VMEM((RW, 16), jnp.int32),
        pltpu.VMEM((NB,), jnp.int32),
        pltpu.HBM((RW, 16), jnp.int32),
    ],
)()


NCH_D = TPW // RT


def _dispatch_body(hid_hbm, pos_hbm, w_hbm, xg_hbm, sw_hbm,
                   rows_v, idx_v, w_v, semf, sems):
    wid = lax.axis_index("s") * NC + lax.axis_index("c")
    tw = wid * TPW

    def fetch(c, b):
        tb = tw + c * RT
        return [
            pltpu.async_copy(pos_hbm.at[pl.ds(tb, RT)], idx_v.at[b, 0], semf),
            pltpu.async_copy(pos_hbm.at[pl.ds(T + tb, RT)], idx_v.at[b, 1], semf),
            pltpu.async_copy(w_hbm.at[pl.ds(tb, RT)], w_v.at[b, 0], semf),
            pltpu.async_copy(w_hbm.at[pl.ds(T + tb, RT)], w_v.at[b, 1], semf),
            pltpu.async_copy(hid_hbm.at[pl.ds(tb, RT)], rows_v.at[b], semf),
        ]

    def scatter(c, b):
        cps = []
        for k in range(K):
            cps.append(pltpu.async_copy(
                rows_v.at[b], xg_hbm.at[idx_v.at[b, k]], sems))
            cps.append(pltpu.async_copy(
                w_v.at[b, k], sw_hbm.at[idx_v.at[b, k]], sems))
        return cps

    fh = {0: fetch(0, 0)}
    sh = {}
    for c in range(NCH_D):
        if c >= 2:
            for cp in sh[c - 2]:
                cp.wait()
        if c + 1 < NCH_D:
            fh[c + 1] = fetch(c + 1, (c + 1) % 3)
        for cp in fh[c]:
            cp.wait()
        sh[c] = scatter(c, c % 3)
    for cp in sh[NCH_D - 2] + sh[NCH_D - 1]:
        cp.wait()


_dispatch = functools.partial(
    pl.kernel,
    _dispatch_body,
    out_type=(jax.ShapeDtypeStruct((P, D), jnp.float32),
              jax.ShapeDtypeStruct((P,), jnp.float32)),
    mesh=plsc.VectorSubcoreMesh(core_axis_name="c", subcore_axis_name="s"),
    scratch_types=[
        pltpu.VMEM((3, RT, D), jnp.float32),
        pltpu.VMEM((3, K, RT), jnp.int32),
        pltpu.VMEM((3, K, RT), jnp.float32),
        pltpu.SemaphoreType.DMA,
        pltpu.SemaphoreType.DMA,
    ],
)()


def _glu_body(be_ref, nab_ref, x_ref, gu_ref, d_ref, w_ref, y_ref):
    @pl.when(pl.program_id(0) < nab_ref[0])
    def _():
        x = x_ref[...].astype(jnp.bfloat16)
        w = w_ref[0, 0][:, None]
        acc = jnp.zeros((B, D), jnp.float32)
        for j in range(J):
            g = lax.dot_general(x, gu_ref[0, pl.ds(j * TI, TI), :].astype(jnp.bfloat16),
                                (((1,), (1,)), ((), ())),
                                preferred_element_type=jnp.float32)
            u = lax.dot_general(x, gu_ref[0, pl.ds(I + j * TI, TI), :].astype(jnp.bfloat16),
                                (((1,), (1,)), ((), ())),
                                preferred_element_type=jnp.float32)
            a = (g * lax.logistic(g) * u * w).astype(jnp.bfloat16)
            acc += lax.dot_general(a, d_ref[0, :, pl.ds(j * TI, TI)].astype(jnp.bfloat16),
                                   (((1,), (1,)), ((), ())),
                                   preferred_element_type=jnp.float32)
        y_ref[...] = acc


def _grouped_glu(xg, gate_up_proj, down_proj, block_expert, nab, slot_weight):
    def bc(b, nab_r):
        return jnp.minimum(b, nab_r[0] - 1)

    grid_spec = pltpu.PrefetchScalarGridSpec(
        num_scalar_prefetch=2,
        grid=(NB,),
        in_specs=[
            pl.BlockSpec((B, D), lambda b, be, na: (bc(b, na), 0)),
            pl.BlockSpec((1, 2 * I, D), lambda b, be, na: (be[bc(b, na)], 0, 0)),
            pl.BlockSpec((1, D, I), lambda b, be, na: (be[bc(b, na)], 0, 0)),
            pl.BlockSpec((1, 1, B), lambda b, be, na: (bc(b, na), 0, 0)),
        ],
        out_specs=pl.BlockSpec((B, D), lambda b, be, na: (bc(b, na), 0)),
    )
    return pl.pallas_call(
        _glu_body,
        grid_spec=grid_spec,
        out_shape=jax.ShapeDtypeStruct((P, D), jnp.float32),
    )(block_expert, nab, xg, gate_up_proj, down_proj, slot_weight)


RTC = 8
NCH_C = TPW // RTC


def _combine_body(y_hbm, pos_hbm, out_hbm,
                  y0_v, y1_v, o_v, idx_v, semg, semo):
    wid = lax.axis_index("s") * NC + lax.axis_index("c")
    tw = wid * TPW
    pltpu.sync_copy(pos_hbm.at[pl.ds(tw, TPW)], idx_v.at[0])
    pltpu.sync_copy(pos_hbm.at[pl.ds(T + tw, TPW)], idx_v.at[1])

    def gath(c, b):
        sl = pl.ds(c * RTC, RTC)
        return [
            pltpu.async_copy(y_hbm.at[idx_v.at[0, sl]], y0_v.at[b], semg),
            pltpu.async_copy(y_hbm.at[idx_v.at[1, sl]], y1_v.at[b], semg),
        ]

    gh = {0: gath(0, 0)}
    oh = {}
    for c in range(NCH_C):
        if c + 1 < NCH_C:
            gh[c + 1] = gath(c + 1, (c + 1) % 2)
        for cp in gh[c]:
            cp.wait()
        if c >= 2:
            oh[c - 2].wait()
        b = c % 2

        def token_body(t, _):
            def col_body(cc, _):
                sl = pl.ds(cc * 16, 16)
                o_v[b, t, sl] = y0_v[b, t, sl] + y1_v[b, t, sl]
                return 0

            lax.fori_loop(0, D // 16, col_body, 0, unroll=8)
            return 0

        lax.fori_loop(0, RTC, token_body, 0)
        oh[c] = pltpu.async_copy(o_v.at[b], out_hbm.at[pl.ds(tw + c * RTC, RTC)],
                                 semo)
    oh[NCH_C - 2].wait()
    oh[NCH_C - 1].wait()


_combine = functools.partial(
    pl.kernel,
    _combine_body,
    out_type=jax.ShapeDtypeStruct((T, D), jnp.float32),
    mesh=plsc.VectorSubcoreMesh(core_axis_name="c", subcore_axis_name="s"),
    scratch_types=[
        pltpu.VMEM((2, RTC, D), jnp.float32),
        pltpu.VMEM((2, RTC, D), jnp.float32),
        pltpu.VMEM((2, RTC, D), jnp.float32),
        pltpu.VMEM((K, TPW), jnp.int32),
        pltpu.SemaphoreType.DMA,
        pltpu.SemaphoreType.DMA,
    ],
)()


@jax.jit
def kernel(hidden_states, top_k_indices, top_k_weights, gate_up_proj, down_proj):
    eT = top_k_indices.T.reshape(-1).astype(jnp.int32)
    wT = top_k_weights.T.reshape(-1)
    pos, block_expert, nab = _route(eT)
    xg, sw = _dispatch(hidden_states, pos, wT)
    y = _grouped_glu(xg, gate_up_proj, down_proj, block_expert, nab,
                     sw.reshape(NB, 1, B))
    return _combine(y, pos)


# speedup vs baseline: 1.2474x; 1.0075x over previous
import functools

import jax
import jax.numpy as jnp
from jax import lax
from jax.experimental import pallas as pl
from jax.experimental.pallas import tpu as pltpu
from jax.experimental.pallas import tpu_sc as plsc

T, D, I, E, K = 4096, 2048, 1024, 16, 2
A = T * K
B = 256
P = A + E * B
NB = P // B
J = 1
TI = I // J

NC, NS = 2, 16
NW = NC * NS
TPW = T // NW
RT = 16

RW = 16
CPW = A // RW
CV = CPW // 16


def _splat(vec, idx):
    return vec.at[jnp.full((16,), idx, jnp.int32)].get(mode="promise_in_bounds")


def _rd_body(e_hbm, hid_hbm, w_hbm, pos_hbm, be_hbm, nab_hbm, xg_hbm, sw_hbm,
             ev_v, rank_v, slot_v, cnt_v, allcnt_v, bev_v,
             rows_v, idx_v, w_v, stage_hbm, slotstage_hbm, semf, sems):
    cid = lax.axis_index("c")
    sid = lax.axis_index("s")
    base_a = sid * CPW
    pltpu.sync_copy(e_hbm.at[pl.ds(base_a, CPW)], ev_v)

    for e in range(E):
        def vec_body(i, base):
            sl = pl.ds(i * 16, 16)
            ev = ev_v[sl]
            m = ev == e
            csum = plsc.cumsum(jnp.where(m, 1, 0))
            rank_v[sl] = jnp.where(m, base + csum - 1, rank_v[sl])
            return base + plsc.all_reduce_population_count(m)

        cnt = lax.fori_loop(0, CV, vec_body, jnp.zeros((16,), jnp.int32))
        lane = lax.iota(jnp.int32, 16)
        cnt_v[...] = jnp.where(lane == e, cnt, cnt_v[...])

    pltpu.sync_copy(cnt_v, stage_hbm.at[cid, sid])
    plsc.subcore_barrier()
    pltpu.sync_copy(stage_hbm.at[cid], allcnt_v)

    gc = jnp.zeros((16,), jnp.int32)
    for w in range(RW):
        gc = gc + allcnt_v[w]
    padded = (gc + (B - 1)) & ~(B - 1)
    ends = plsc.cumsum(padded)
    base_w = ends - padded
    for w in range(RW):
        base_w = base_w + jnp.where(w < sid, allcnt_v[w], 0)

    def slot_body(i, _):
        sl = pl.ds(i * 16, 16)
        ev = ev_v[sl]
        bases = base_w.at[ev].get(mode="promise_in_bounds")
        slot_v[sl] = bases + rank_v[sl]
        return 0

    lax.fori_loop(0, CV, slot_body, 0)
    pltpu.sync_copy(slot_v, slotstage_hbm.at[cid, pl.ds(base_a, CPW)])

    @pl.when(cid == 0)
    def _():
        pltpu.sync_copy(slot_v, pos_hbm.at[pl.ds(base_a, CPW)])

    @pl.when((cid == 0) & (sid == 0))
    def _():
        for q in range(NB // 16):
            bs = (lax.iota(jnp.int32, 16) + q * 16) * B
            bev = jnp.zeros((16,), jnp.int32)
            for e in range(E):
                bev = bev + jnp.where(bs >= _splat(ends, e), 1, 0)
            bev_v[pl.ds(q * 16, 16)] = jnp.minimum(bev, E - 1)
        pltpu.sync_copy(bev_v, be_hbm)
        cnt_v[...] = lax.shift_right_logical(_splat(ends, E - 1), B.bit_length() - 1)
        pltpu.sync_copy(cnt_v, nab_hbm)

    plsc.subcore_barrier()

    tw = (sid * NC + cid) * TPW

    def fetch(c, b):
        tb = tw + c * RT
        return [
            pltpu.async_copy(slotstage_hbm.at[cid, pl.ds(tb, RT)], idx_v.at[b, 0], semf),
            pltpu.async_copy(slotstage_hbm.at[cid, pl.ds(T + tb, RT)], idx_v.at[b, 1], semf),
            pltpu.async_copy(w_hbm.at[pl.ds(tb, RT)], w_v.at[b, 0], semf),
            pltpu.async_copy(w_hbm.at[pl.ds(T + tb, RT)], w_v.at[b, 1], semf),
            pltpu.async_copy(hid_hbm.at[pl.ds(tb, RT)], rows_v.at[b], semf),
        ]

    def scatter(c, b):
        cps = []
        for k in range(K):
            cps.append(pltpu.async_copy(
                rows_v.at[b], xg_hbm.at[idx_v.at[b, k]], sems))
            cps.append(pltpu.async_copy(
                w_v.at[b, k], sw_hbm.at[idx_v.at[b, k]], sems))
        return cps

    fh = {0: fetch(0, 0)}
    sh = {}
    for c in range(NCH_D):
        if c >= 2:
            for cp in sh[c - 2]:
                cp.wait()
        if c + 1 < NCH_D:
            fh[c + 1] = fetch(c + 1, (c + 1) % 3)
        for cp in fh[c]:
            cp.wait()
        sh[c] = scatter(c, c % 3)
    for cp in sh[NCH_D - 2] + sh[NCH_D - 1]:
        cp.wait()


NCH_D = TPW // RT

_route_dispatch = functools.partial(
    pl.kernel,
    _rd_body,
    out_type=(jax.ShapeDtypeStruct((A,), jnp.int32),
              jax.ShapeDtypeStruct((NB,), jnp.int32),
              jax.ShapeDtypeStruct((16,), jnp.int32),
              jax.ShapeDtypeStruct((P, D), jnp.float32),
              jax.ShapeDtypeStruct((P,), jnp.float32)),
    mesh=plsc.VectorSubcoreMesh(core_axis_name="c", subcore_axis_name="s"),
    compiler_params=pltpu.CompilerParams(needs_layout_passes=False),
    scratch_types=[
        pltpu.VMEM((CPW,), jnp.int32),
        pltpu.VMEM((CPW,), jnp.int32),
        pltpu.VMEM((CPW,), jnp.int32),
        pltpu.VMEM((16,), jnp.int32),
        pltpu.VMEM((RW, 16), jnp.int32),
        pltpu.VMEM((NB,), jnp.int32),
        pltpu.VMEM((3, RT, D), jnp.float32),
        pltpu.VMEM((3, K, RT), jnp.int32),
        pltpu.VMEM((3, K, RT), jnp.float32),
        pltpu.HBM((NC, RW, 16), jnp.int32),
        pltpu.HBM((NC, A), jnp.int32),
        pltpu.SemaphoreType.DMA,
        pltpu.SemaphoreType.DMA,
    ],
)()


def _glu_body(be_ref, nab_ref, x_ref, gu_ref, d_ref, w_ref, y_ref):
    @pl.when(pl.program_id(0) < nab_ref[0])
    def _():
        x = x_ref[...].astype(jnp.bfloat16)
        w = w_ref[0, 0][:, None]
        acc = jnp.zeros((B, D), jnp.float32)
        for j in range(J):
            g = lax.dot_general(x, gu_ref[0, pl.ds(j * TI, TI), :].astype(jnp.bfloat16),
                                (((1,), (1,)), ((), ())),
                                preferred_element_type=jnp.float32)
            u = lax.dot_general(x, gu_ref[0, pl.ds(I + j * TI, TI), :].astype(jnp.bfloat16),
                                (((1,), (1,)), ((), ())),
                                preferred_element_type=jnp.float32)
            a = (g * lax.logistic(g) * u * w).astype(jnp.bfloat16)
            acc += lax.dot_general(a, d_ref[0, :, pl.ds(j * TI, TI)].astype(jnp.bfloat16),
                                   (((1,), (1,)), ((), ())),
                                   preferred_element_type=jnp.float32)
        y_ref[...] = acc


def _grouped_glu(xg, gate_up_proj, down_proj, block_expert, nab, slot_weight):
    def bc(b, nab_r):
        return jnp.minimum(b, nab_r[0] - 1)

    grid_spec = pltpu.PrefetchScalarGridSpec(
        num_scalar_prefetch=2,
        grid=(NB,),
        in_specs=[
            pl.BlockSpec((B, D), lambda b, be, na: (bc(b, na), 0)),
            pl.BlockSpec((1, 2 * I, D), lambda b, be, na: (be[bc(b, na)], 0, 0)),
            pl.BlockSpec((1, D, I), lambda b, be, na: (be[bc(b, na)], 0, 0)),
            pl.BlockSpec((1, 1, B), lambda b, be, na: (bc(b, na), 0, 0)),
        ],
        out_specs=pl.BlockSpec((B, D), lambda b, be, na: (bc(b, na), 0)),
    )
    return pl.pallas_call(
        _glu_body,
        grid_spec=grid_spec,
        out_shape=jax.ShapeDtypeStruct((P, D), jnp.float32),
    )(block_expert, nab, xg, gate_up_proj, down_proj, slot_weight)


RTC = 8
NCH_C = TPW // RTC


def _combine_body(y_hbm, pos_hbm, out_hbm,
                  y0_v, y1_v, o_v, idx_v, semg, semo):
    wid = lax.axis_index("s") * NC + lax.axis_index("c")
    tw = wid * TPW
    pltpu.sync_copy(pos_hbm.at[pl.ds(tw, TPW)], idx_v.at[0])
    pltpu.sync_copy(pos_hbm.at[pl.ds(T + tw, TPW)], idx_v.at[1])

    def gath(c, b):
        sl = pl.ds(c * RTC, RTC)
        return [
            pltpu.async_copy(y_hbm.at[idx_v.at[0, sl]], y0_v.at[b], semg),
            pltpu.async_copy(y_hbm.at[idx_v.at[1, sl]], y1_v.at[b], semg),
        ]

    gh = {0: gath(0, 0)}
    oh = {}
    for c in range(NCH_C):
        if c + 1 < NCH_C:
            gh[c + 1] = gath(c + 1, (c + 1) % 2)
        for cp in gh[c]:
            cp.wait()
        if c >= 2:
            oh[c - 2].wait()
        b = c % 2

        def token_body(t, _):
            def col_body(cc, _):
                sl = pl.ds(cc * 16, 16)
                o_v[b, t, sl] = y0_v[b, t, sl] + y1_v[b, t, sl]
                return 0

            lax.fori_loop(0, D // 16, col_body, 0, unroll=8)
            return 0

        lax.fori_loop(0, RTC, token_body, 0)
        oh[c] = pltpu.async_copy(o_v.at[b], out_hbm.at[pl.ds(tw + c * RTC, RTC)],
                                 semo)
    oh[NCH_C - 2].wait()
    oh[NCH_C - 1].wait()


_combine = functools.partial(
    pl.kernel,
    _combine_body,
    out_type=jax.ShapeDtypeStruct((T, D), jnp.float32),
    mesh=plsc.VectorSubcoreMesh(core_axis_name="c", subcore_axis_name="s"),
    scratch_types=[
        pltpu.VMEM((2, RTC, D), jnp.float32),
        pltpu.VMEM((2, RTC, D), jnp.float32),
        pltpu.VMEM((2, RTC, D), jnp.float32),
        pltpu.VMEM((K, TPW), jnp.int32),
        pltpu.SemaphoreType.DMA,
        pltpu.SemaphoreType.DMA,
    ],
)()


@jax.jit
def kernel(hidden_states, top_k_indices, top_k_weights, gate_up_proj, down_proj):
    eT = top_k_indices.T.reshape(-1).astype(jnp.int32)
    wT = top_k_weights.T.reshape(-1)
    pos, block_expert, nab, xg, sw = _route_dispatch(eT, hidden_states, wT)
    y = _grouped_glu(xg, gate_up_proj, down_proj, block_expert, nab,
                     sw.reshape(NB, 1, B))
    return _combine(y, pos)


# speedup vs baseline: 1.2477x; 1.0002x over previous
import functools

import jax
import jax.numpy as jnp
from jax import lax
from jax.experimental import pallas as pl
from jax.experimental.pallas import tpu as pltpu
from jax.experimental.pallas import tpu_sc as plsc

T, D, I, E, K = 4096, 2048, 1024, 16, 2
A = T * K
B = 256
P = A + E * B
NB = P // B
J = 1
TI = I // J

NC, NS = 2, 16
NW = NC * NS
TPW = T // NW
RT = 16

RW = 16
CPW = A // RW
CV = CPW // 16


def _splat(vec, idx):
    return vec.at[jnp.full((16,), idx, jnp.int32)].get(mode="promise_in_bounds")


def _rd_body(e_hbm, hid_hbm, w_hbm, pos_hbm, be_hbm, nab_hbm, xg_hbm, sw_hbm,
             ev_v, rank_v, slot_v, cnt_v, allcnt_v, bev_v,
             rows_v, idx_v, w_v, stage_hbm, slotstage_hbm, semf, sems):
    cid = lax.axis_index("c")
    sid = lax.axis_index("s")
    base_a = sid * CPW
    pltpu.sync_copy(e_hbm.at[pl.ds(base_a, CPW)], ev_v)

    for e in range(E):
        def vec_body(i, base):
            sl = pl.ds(i * 16, 16)
            ev = ev_v[sl]
            m = ev == e
            csum = plsc.cumsum(jnp.where(m, 1, 0))
            rank_v[sl] = jnp.where(m, base + csum - 1, rank_v[sl])
            return base + plsc.all_reduce_population_count(m)

        cnt = lax.fori_loop(0, CV, vec_body, jnp.zeros((16,), jnp.int32))
        lane = lax.iota(jnp.int32, 16)
        cnt_v[...] = jnp.where(lane == e, cnt, cnt_v[...])

    pltpu.sync_copy(cnt_v, stage_hbm.at[cid, sid])
    plsc.subcore_barrier()
    pltpu.sync_copy(stage_hbm.at[cid], allcnt_v)

    gc = jnp.zeros((16,), jnp.int32)
    for w in range(RW):
        gc = gc + allcnt_v[w]
    padded = (gc + (B - 1)) & ~(B - 1)
    ends = plsc.cumsum(padded)
    base_w = ends - padded
    for w in range(RW):
        base_w = base_w + jnp.where(w < sid, allcnt_v[w], 0)

    def slot_body(i, _):
        sl = pl.ds(i * 16, 16)
        ev = ev_v[sl]
        bases = base_w.at[ev].get(mode="promise_in_bounds")
        slot_v[sl] = bases + rank_v[sl]
        return 0

    lax.fori_loop(0, CV, slot_body, 0)
    pltpu.sync_copy(slot_v, slotstage_hbm.at[cid, pl.ds(base_a, CPW)])

    @pl.when(cid == 0)
    def _():
        pltpu.sync_copy(slot_v, pos_hbm.at[pl.ds(base_a, CPW)])

    @pl.when((cid == 0) & (sid == 0))
    def _():
        for q in range(NB // 16):
            bs = (lax.iota(jnp.int32, 16) + q * 16) * B
            bev = jnp.zeros((16,), jnp.int32)
            for e in range(E):
                bev = bev + jnp.where(bs >= _splat(ends, e), 1, 0)
            bev_v[pl.ds(q * 16, 16)] = jnp.minimum(bev, E - 1)
        pltpu.sync_copy(bev_v, be_hbm)
        cnt_v[...] = lax.shift_right_logical(_splat(ends, E - 1), B.bit_length() - 1)
        pltpu.sync_copy(cnt_v, nab_hbm)

    plsc.subcore_barrier()

    tw = (sid * NC + cid) * TPW

    def fetch(c, b):
        tb = tw + c * RT
        return [
            pltpu.async_copy(slotstage_hbm.at[cid, pl.ds(tb, RT)], idx_v.at[b, 0], semf),
            pltpu.async_copy(slotstage_hbm.at[cid, pl.ds(T + tb, RT)], idx_v.at[b, 1], semf),
            pltpu.async_copy(w_hbm.at[pl.ds(tb, RT)], w_v.at[b, 0], semf),
            pltpu.async_copy(w_hbm.at[pl.ds(T + tb, RT)], w_v.at[b, 1], semf),
            pltpu.async_copy(hid_hbm.at[pl.ds(tb, RT)], rows_v.at[b], semf),
        ]

    def scatter(c, b):
        cps = []
        for k in range(K):
            cps.append(pltpu.async_copy(
                rows_v.at[b], xg_hbm.at[idx_v.at[b, k]], sems))
            cps.append(pltpu.async_copy(
                w_v.at[b, k], sw_hbm.at[idx_v.at[b, k]], sems))
        return cps

    fh = {0: fetch(0, 0)}
    sh = {}
    for c in range(NCH_D):
        if c >= 2:
            for cp in sh[c - 2]:
                cp.wait()
        if c + 1 < NCH_D:
            fh[c + 1] = fetch(c + 1, (c + 1) % 3)
        for cp in fh[c]:
            cp.wait()
        sh[c] = scatter(c, c % 3)
    for cp in sh[NCH_D - 2] + sh[NCH_D - 1]:
        cp.wait()


NCH_D = TPW // RT

_route_dispatch = functools.partial(
    pl.kernel,
    _rd_body,
    out_type=(jax.ShapeDtypeStruct((A,), jnp.int32),
              jax.ShapeDtypeStruct((NB,), jnp.int32),
              jax.ShapeDtypeStruct((16,), jnp.int32),
              jax.ShapeDtypeStruct((P, D), jnp.float32),
              jax.ShapeDtypeStruct((P,), jnp.float32)),
    mesh=plsc.VectorSubcoreMesh(core_axis_name="c", subcore_axis_name="s"),
    compiler_params=pltpu.CompilerParams(needs_layout_passes=False),
    scratch_types=[
        pltpu.VMEM((CPW,), jnp.int32),
        pltpu.VMEM((CPW,), jnp.int32),
        pltpu.VMEM((CPW,), jnp.int32),
        pltpu.VMEM((16,), jnp.int32),
        pltpu.VMEM((RW, 16), jnp.int32),
        pltpu.VMEM((NB,), jnp.int32),
        pltpu.VMEM((3, RT, D), jnp.float32),
        pltpu.VMEM((3, K, RT), jnp.int32),
        pltpu.VMEM((3, K, RT), jnp.float32),
        pltpu.HBM((NC, RW, 16), jnp.int32),
        pltpu.HBM((NC, A), jnp.int32),
        pltpu.SemaphoreType.DMA,
        pltpu.SemaphoreType.DMA,
    ],
)()


def _glu_body(be_ref, nab_ref, x_ref, gu_ref, d_ref, w_ref, y_ref):
    @pl.when(pl.program_id(0) < nab_ref[0])
    def _():
        x = x_ref[...].astype(jnp.bfloat16)
        w = w_ref[0, 0][:, None]
        acc = jnp.zeros((B, D), jnp.float32)
        for j in range(J):
            g = lax.dot_general(x, gu_ref[0, pl.ds(j * TI, TI), :].astype(jnp.bfloat16),
                                (((1,), (1,)), ((), ())),
                                preferred_element_type=jnp.float32)
            u = lax.dot_general(x, gu_ref[0, pl.ds(I + j * TI, TI), :].astype(jnp.bfloat16),
                                (((1,), (1,)), ((), ())),
                                preferred_element_type=jnp.float32)
            a = (g * lax.logistic(g) * u * w).astype(jnp.bfloat16)
            acc += lax.dot_general(a, d_ref[0, :, pl.ds(j * TI, TI)].astype(jnp.bfloat16),
                                   (((1,), (1,)), ((), ())),
                                   preferred_element_type=jnp.float32)
        y_ref[...] = acc


def _grouped_glu(xg, gate_up_proj, down_proj, block_expert, nab, slot_weight):
    def bc(b, nab_r):
        return jnp.minimum(b, nab_r[0] - 1)

    grid_spec = pltpu.PrefetchScalarGridSpec(
        num_scalar_prefetch=2,
        grid=(NB,),
        in_specs=[
            pl.BlockSpec((B, D), lambda b, be, na: (bc(b, na), 0)),
            pl.BlockSpec((1, 2 * I, D), lambda b, be, na: (be[bc(b, na)], 0, 0)),
            pl.BlockSpec((1, D, I), lambda b, be, na: (be[bc(b, na)], 0, 0)),
            pl.BlockSpec((1, 1, B), lambda b, be, na: (bc(b, na), 0, 0)),
        ],
        out_specs=pl.BlockSpec((B, D), lambda b, be, na: (bc(b, na), 0)),
    )
    return pl.pallas_call(
        _glu_body,
        grid_spec=grid_spec,
        out_shape=jax.ShapeDtypeStruct((P, D), jnp.float32),
    )(block_expert, nab, xg, gate_up_proj, down_proj, slot_weight)


RTC = 8
NCH_C = TPW // RTC


def _combine_body(y_hbm, pos_hbm, out_hbm,
                  y0_v, y1_v, o_v, idx_v, semg, semo):
    wid = lax.axis_index("s") * NC + lax.axis_index("c")
    tw = wid * TPW
    pltpu.sync_copy(pos_hbm.at[pl.ds(tw, TPW)], idx_v.at[0])
    pltpu.sync_copy(pos_hbm.at[pl.ds(T + tw, TPW)], idx_v.at[1])

    def gath(c, b):
        sl = pl.ds(c * RTC, RTC)
        return [
            pltpu.async_copy(y_hbm.at[idx_v.at[0, sl]], y0_v.at[b], semg),
            pltpu.async_copy(y_hbm.at[idx_v.at[1, sl]], y1_v.at[b], semg),
        ]

    gh = {0: gath(0, 0)}
    oh = {}
    for c in range(NCH_C):
        if c + 1 < NCH_C:
            gh[c + 1] = gath(c + 1, (c + 1) % 2)
        for cp in gh[c]:
            cp.wait()
        if c >= 2:
            oh[c - 2].wait()
        b = c % 2

        def token_body(t, _):
            def col_body(cc, _):
                sl = pl.ds(cc * 16, 16)
                o_v[b, t, sl] = y0_v[b, t, sl] + y1_v[b, t, sl]
                return 0

            lax.fori_loop(0, D // 16, col_body, 0, unroll=16)
            return 0

        lax.fori_loop(0, RTC, token_body, 0)
        oh[c] = pltpu.async_copy(o_v.at[b], out_hbm.at[pl.ds(tw + c * RTC, RTC)],
                                 semo)
    oh[NCH_C - 2].wait()
    oh[NCH_C - 1].wait()


_combine = functools.partial(
    pl.kernel,
    _combine_body,
    out_type=jax.ShapeDtypeStruct((T, D), jnp.float32),
    mesh=plsc.VectorSubcoreMesh(core_axis_name="c", subcore_axis_name="s"),
    scratch_types=[
        pltpu.VMEM((2, RTC, D), jnp.float32),
        pltpu.VMEM((2, RTC, D), jnp.float32),
        pltpu.VMEM((2, RTC, D), jnp.float32),
        pltpu.VMEM((K, TPW), jnp.int32),
        pltpu.SemaphoreType.DMA,
        pltpu.SemaphoreType.DMA,
    ],
)()


@jax.jit
def kernel(hidden_states, top_k_indices, top_k_weights, gate_up_proj, down_proj):
    eT = top_k_indices.T.reshape(-1).astype(jnp.int32)
    wT = top_k_weights.T.reshape(-1)
    pos, block_expert, nab, xg, sw = _route_dispatch(eT, hidden_states, wT)
    y = _grouped_glu(xg, gate_up_proj, down_proj, block_expert, nab,
                     sw.reshape(NB, 1, B))
    return _combine(y, pos)
